# Initial kernel scaffold; baseline (speedup 1.0000x reference)
import jax
import jax.numpy as jnp
from jax.experimental import pallas as pl


def kernel(params, n_coor, n_info, dist):
    raise NotImplementedError("write your pallas kernel here")


# speedup vs baseline: 2.3588x; 2.3588x over previous
import math

import jax
import jax.numpy as jnp
from jax.experimental import pallas as pl

_B, _N, _H, _L, _K = 8, 100, 128, 2, 10
_NP = 128
_TI = 20
_NT = _N // _TI
_R = _TI * _N


def _ln(x, g, b):
    m = jnp.mean(x, axis=-1, keepdims=True)
    d = x - m
    v = jnp.mean(d * d, axis=-1, keepdims=True)
    return d / jnp.sqrt(v + 1e-5) * g + b


def _topk_kernel(dist_ref, adj_ref, mask_ref):
    w = dist_ref[0]
    col = jax.lax.broadcasted_iota(jnp.int32, (_NP, _NP), 1)
    row = jax.lax.broadcasted_iota(jnp.int32, (_NP, _NP), 0)
    adj = jnp.zeros((_NP, _NP), jnp.float32)
    msk = jnp.zeros((_NP, _NP), jnp.float32)
    for t in range(_K + 1):
        m = jnp.min(w, axis=-1, keepdims=True)
        cand = w == m
        idx = jnp.min(jnp.where(cand, col, jnp.int32(2**30)), axis=-1,
                      keepdims=True)
        hit = col == idx
        adj = adj + hit.astype(jnp.float32)
        if t >= 1:
            msk = msk + hit.astype(jnp.float32)
        w = jnp.where(hit, jnp.float32(jnp.inf), w)
    adj = jnp.where(row == col, jnp.float32(-1.0), adj)
    adj_ref[0] = adj
    mask_ref[0] = msk


def _node_kernel(in8_ref, mask_ref, wx_ref, mats_ref, vecs_ref,
                 hn_ref, e2a_ref, e3a_ref, e2b_ref, e3b_ref):
    f32 = jnp.float32
    in8 = in8_ref[0]
    x0 = jnp.maximum(jnp.dot(in8, wx_ref[0], preferred_element_type=f32)
                     + vecs_ref[0], 0.0)
    xi = jnp.maximum(jnp.dot(in8, wx_ref[1], preferred_element_type=f32)
                     + vecs_ref[1], 0.0)
    rid = jax.lax.broadcasted_iota(jnp.int32, (_NP, _H), 0)
    x = jnp.where(rid == 0, x0, xi)
    h = jnp.dot(x, mats_ref[0], preferred_element_type=f32) + vecs_ref[2]
    mask = mask_ref[0]
    e2refs = (e2a_ref, e2b_ref)
    e3refs = (e3a_ref, e3b_ref)
    for l in range(_L):
        mb = 1 + 9 * l
        vb = 3 + 12 * l
        q = jnp.dot(h, mats_ref[mb + 0], preferred_element_type=f32) + vecs_ref[vb + 0]
        k = jnp.dot(h, mats_ref[mb + 1], preferred_element_type=f32) + vecs_ref[vb + 1]
        v = jnp.dot(h, mats_ref[mb + 2], preferred_element_type=f32) + vecs_ref[vb + 2]
        s = jax.lax.dot_general(q, k, (((1,), (1,)), ((), ())),
                                preferred_element_type=f32) * (1.0 / math.sqrt(_H))
        s = jnp.where(mask > 0.5, s, f32(-1e30))
        smax = jnp.max(s, axis=-1, keepdims=True)
        e = jnp.exp(s - smax)
        att = e / jnp.sum(e, axis=-1, keepdims=True)
        hatt = jnp.dot(att, v, preferred_element_type=f32)
        hagg = h + jnp.maximum(
            jnp.dot(hatt, mats_ref[mb + 3], preferred_element_type=f32)
            + vecs_ref[vb + 3], 0.0)
        hagg = _ln(hagg, vecs_ref[vb + 4], vecs_ref[vb + 5])
        c = jnp.dot(h, mats_ref[mb + 4], preferred_element_type=f32) + vecs_ref[vb + 6]
        hcom = hagg + jnp.maximum(
            jnp.dot(c, mats_ref[mb + 5], preferred_element_type=f32)
            + jnp.dot(hagg, mats_ref[mb + 6], preferred_element_type=f32)
            + vecs_ref[vb + 7], 0.0)
        hn_next = _ln(hcom, vecs_ref[vb + 8], vecs_ref[vb + 9])
        e2refs[l][0] = jnp.dot(h, mats_ref[mb + 7], preferred_element_type=f32) + vecs_ref[vb + 10]
        e3refs[l][0] = jnp.dot(h, mats_ref[mb + 8], preferred_element_type=f32) + vecs_ref[vb + 11]
        h = hn_next
    hn_ref[0] = h


def _edge_kernel(dcol_ref, acol_ref, e2a_ref, e3a_ref, e2b_ref, e3b_ref,
                 mats_ref, vecs_ref, out_ref):
    f32 = jnp.float32
    d = dcol_ref[0]
    a = acol_ref[0]
    y = jnp.maximum(d * vecs_ref[0] + a * vecs_ref[1] + vecs_ref[2], 0.0)
    he = jnp.dot(y, mats_ref[0], preferred_element_type=f32) + vecs_ref[3]
    e2refs = (e2a_ref, e2b_ref)
    e3refs = (e3a_ref, e3b_ref)
    for l in range(_L):
        mb = 1 + 5 * l
        vb = 4 + 8 * l
        e1 = jnp.dot(he, mats_ref[mb + 0], preferred_element_type=f32) + vecs_ref[vb + 0]
        e2 = e2refs[l][0]
        e3 = e3refs[l][0]
        s = (e1.reshape(_TI, _N, _H) + e2[:, None, :]
             + e3[None, :, :]).reshape(_R, _H)
        t = jnp.maximum(jnp.dot(s, mats_ref[mb + 1], preferred_element_type=f32)
                        + vecs_ref[vb + 1], 0.0)
        hagg = _ln(he + t, vecs_ref[vb + 2], vecs_ref[vb + 3])
        c = jnp.dot(he, mats_ref[mb + 2], preferred_element_type=f32) + vecs_ref[vb + 4]
        hcom = hagg + jnp.maximum(
            jnp.dot(c, mats_ref[mb + 3], preferred_element_type=f32)
            + jnp.dot(hagg, mats_ref[mb + 4], preferred_element_type=f32)
            + vecs_ref[vb + 5], 0.0)
        he = _ln(hcom, vecs_ref[vb + 6], vecs_ref[vb + 7])
    out_ref[0] = he.reshape(_TI, _N, _H)


def _run(params, n_coor, n_info, dist, interpret=False):
    f32 = jnp.float32
    p = params

    in8 = jnp.zeros((_B, _NP, 8), f32)
    in8 = in8.at[:, :_N, 0:2].set(n_coor)
    in8 = in8.at[:, :_N, 2:5].set(n_info)
    distp = jnp.pad(dist, ((0, 0), (0, _NP - _N), (0, _NP - _N)),
                    constant_values=jnp.inf)

    wx0 = jnp.zeros((8, _H), f32).at[0:2, :].set(p["W1"]["w"].T)
    wxi = jnp.zeros((8, _H), f32)
    wxi = wxi.at[0:2, 0:_H // 2].set(p["W2"]["w"].T)
    wxi = wxi.at[2:5, _H // 2:].set(p["W3"]["w"].T)
    wx = jnp.stack([wx0, wxi])
    nmats = [p["init_n"]["w"].T]
    nvecs = [p["W1"]["b"],
             jnp.concatenate([p["W2"]["b"], p["W3"]["b"]]),
             p["init_n"]["b"]]
    for lp in p["layers"]:
        nmats += [lp["attn_q"]["w"].T, lp["attn_k"]["w"].T, lp["attn_v"]["w"].T,
                  lp["W_node_agg"]["w"].T, lp["V_node_com"]["w"].T,
                  lp["V_node"]["w"][:, :_H].T, lp["V_node"]["w"][:, _H:].T,
                  lp["W_edge_agg_2"]["w"].T, lp["W_edge_agg_3"]["w"].T]
        nvecs += [lp["attn_q"]["b"], lp["attn_k"]["b"], lp["attn_v"]["b"],
                  lp["W_node_agg"]["b"], lp["ln_na"]["g"], lp["ln_na"]["b"],
                  lp["V_node_com"]["b"], lp["V_node"]["b"],
                  lp["ln_nc"]["g"], lp["ln_nc"]["b"],
                  lp["W_edge_agg_2"]["b"], lp["W_edge_agg_3"]["b"]]
    nmats = jnp.stack(nmats)
    nvecs = jnp.stack(nvecs)[:, None, :]

    half = _H // 2
    zeros_h = jnp.zeros((half,), f32)
    emats = [p["init_e"]["w"].T]
    evecs = [jnp.concatenate([p["W4"]["w"][:, 0], zeros_h]),
             jnp.concatenate([zeros_h, p["W5"]["w"][:, 0]]),
             jnp.concatenate([p["W4"]["b"], p["W5"]["b"]]),
             p["init_e"]["b"]]
    for lp in p["layers"]:
        emats += [lp["W_edge_agg_1"]["w"].T, lp["W_edge_agg"]["w"].T,
                  lp["V_edge_com"]["w"].T,
                  lp["V_edge"]["w"][:, :_H].T, lp["V_edge"]["w"][:, _H:].T]
        evecs += [lp["W_edge_agg_1"]["b"], lp["W_edge_agg"]["b"],
                  lp["ln_ea"]["g"], lp["ln_ea"]["b"],
                  lp["V_edge_com"]["b"], lp["V_edge"]["b"],
                  lp["ln_ec"]["g"], lp["ln_ec"]["b"]]
    emats = jnp.stack(emats)
    evecs = jnp.stack(evecs)[:, None, :]

    adj_p, mask_p = pl.pallas_call(
        _topk_kernel,
        grid=(_B,),
        in_specs=[pl.BlockSpec((1, _NP, _NP), lambda b: (b, 0, 0))],
        out_specs=[pl.BlockSpec((1, _NP, _NP), lambda b: (b, 0, 0))] * 2,
        out_shape=[jax.ShapeDtypeStruct((_B, _NP, _NP), f32)] * 2,
        interpret=interpret,
    )(distp)

    nm, nv = nmats.shape[0], nvecs.shape[0]
    hn_p, e2a, e3a, e2b, e3b = pl.pallas_call(
        _node_kernel,
        grid=(_B,),
        in_specs=[
            pl.BlockSpec((1, _NP, 8), lambda b: (b, 0, 0)),
            pl.BlockSpec((1, _NP, _NP), lambda b: (b, 0, 0)),
            pl.BlockSpec((2, 8, _H), lambda b: (0, 0, 0)),
            pl.BlockSpec((nm, _H, _H), lambda b: (0, 0, 0)),
            pl.BlockSpec((nv, 1, _H), lambda b: (0, 0, 0)),
        ],
        out_specs=[pl.BlockSpec((1, _NP, _H), lambda b: (b, 0, 0))] * 5,
        out_shape=[jax.ShapeDtypeStruct((_B, _NP, _H), f32)] * 5,
        interpret=interpret,
    )(in8, mask_p, wx, nmats, nvecs)

    dcol = dist.reshape(_B * _NT, _R, 1)
    acol = adj_p[:, :_N, :_N].reshape(_B * _NT, _R, 1)
    e2a_r = e2a[:, :_N, :].reshape(_B * _NT, _TI, _H)
    e2b_r = e2b[:, :_N, :].reshape(_B * _NT, _TI, _H)
    e3a_r = e3a[:, :_N, :]
    e3b_r = e3b[:, :_N, :]
    em, ev = emats.shape[0], evecs.shape[0]
    he = pl.pallas_call(
        _edge_kernel,
        grid=(_B, _NT),
        in_specs=[
            pl.BlockSpec((1, _R, 1), lambda b, t: (b * _NT + t, 0, 0)),
            pl.BlockSpec((1, _R, 1), lambda b, t: (b * _NT + t, 0, 0)),
            pl.BlockSpec((1, _TI, _H), lambda b, t: (b * _NT + t, 0, 0)),
            pl.BlockSpec((1, _N, _H), lambda b, t: (b, 0, 0)),
            pl.BlockSpec((1, _TI, _H), lambda b, t: (b * _NT + t, 0, 0)),
            pl.BlockSpec((1, _N, _H), lambda b, t: (b, 0, 0)),
            pl.BlockSpec((em, _H, _H), lambda b, t: (0, 0, 0)),
            pl.BlockSpec((ev, 1, _H), lambda b, t: (0, 0, 0)),
        ],
        out_specs=pl.BlockSpec((1, _TI, _N, _H), lambda b, t: (b * _NT + t, 0, 0, 0)),
        out_shape=jax.ShapeDtypeStruct((_B * _NT, _TI, _N, _H), f32),
        interpret=interpret,
    )(dcol, acol, e2a_r, e3a_r, e2b_r, e3b_r, emats, evecs)

    return hn_p[:, :_N, :], he.reshape(_B, _N, _N, _H)


def kernel(params, n_coor, n_info, dist):
    return _run(params, n_coor, n_info, dist)


# speedup vs baseline: 3.3471x; 1.4190x over previous
import math

import jax
import jax.numpy as jnp
from jax.experimental import pallas as pl

_B, _N, _H, _L, _K = 8, 100, 128, 2, 10
_TI = 20
_NT = _N // _TI
_R = _TI * _N
_F32 = jnp.float32


def _dgt(x, w):
    return jax.lax.dot_general(x, w, (((1,), (1,)), ((), ())),
                               preferred_element_type=_F32)


def _ln(x, g, b):
    m = jnp.mean(x, axis=-1, keepdims=True)
    d = x - m
    v = jnp.mean(d * d, axis=-1, keepdims=True)
    return d / jnp.sqrt(v + 1e-5) * g + b


def _prep_kernel(dist_ref, coor_ref, info_ref,
                 w1_ref, w2_ref, w3_ref, wi_ref,
                 wq0_ref, wk0_ref, wv0_ref, wna0_ref, wnc0_ref, wvn0_ref,
                 we20_ref, we30_ref,
                 wq1_ref, wk1_ref, wv1_ref, wna1_ref, wnc1_ref, wvn1_ref,
                 we21_ref, we31_ref,
                 vec_ref,
                 adj_ref, distr_ref, hn_ref,
                 e2a_ref, e3a_ref, e2b_ref, e3b_ref):
    w = dist_ref[0]
    col = jax.lax.broadcasted_iota(jnp.int32, (_N, _N), 1)
    row = jax.lax.broadcasted_iota(jnp.int32, (_N, _N), 0)
    adj = jnp.zeros((_N, _N), _F32)
    msk = jnp.zeros((_N, _N), _F32)
    for t in range(_K + 1):
        m = jnp.min(w, axis=-1, keepdims=True)
        cand = w == m
        idx = jnp.min(jnp.where(cand, col, jnp.int32(2**30)), axis=-1,
                      keepdims=True)
        hit = col == idx
        adj = adj + hit.astype(_F32)
        if t >= 1:
            msk = msk + hit.astype(_F32)
        w = jnp.where(hit, _F32(jnp.inf), w)
    adj = jnp.where(row == col, _F32(-1.0), adj)
    adj_ref[0] = adj.reshape(_NT, _TI, _N)
    distr_ref[0] = dist_ref[0].reshape(_NT, _TI, _N)

    coor = coor_ref[0]
    info = info_ref[0]
    x0 = jnp.maximum(_dgt(coor, w1_ref[...]) + vec_ref[0], 0.0)
    xi = jnp.maximum(
        jnp.concatenate([_dgt(coor, w2_ref[...]), _dgt(info, w3_ref[...])],
                        axis=-1) + vec_ref[1], 0.0)
    rid = jax.lax.broadcasted_iota(jnp.int32, (_N, _H), 0)
    x = jnp.where(rid == 0, x0, xi)
    h = _dgt(x, wi_ref[...]) + vec_ref[2]
    mats = ((wq0_ref, wk0_ref, wv0_ref, wna0_ref, wnc0_ref, wvn0_ref,
             we20_ref, we30_ref),
            (wq1_ref, wk1_ref, wv1_ref, wna1_ref, wnc1_ref, wvn1_ref,
             we21_ref, we31_ref))
    e2refs = (e2a_ref, e2b_ref)
    e3refs = (e3a_ref, e3b_ref)
    for l in range(_L):
        wq, wk, wv, wna, wnc, wvn, we2, we3 = mats[l]
        vb = 3 + 12 * l
        q = _dgt(h, wq[...]) + vec_ref[vb + 0]
        k = _dgt(h, wk[...]) + vec_ref[vb + 1]
        v = _dgt(h, wv[...]) + vec_ref[vb + 2]
        s = _dgt(q, k) * (1.0 / math.sqrt(_H))
        s = jnp.where(msk > 0.5, s, _F32(-1e30))
        smax = jnp.max(s, axis=-1, keepdims=True)
        e = jnp.exp(s - smax)
        att = e / jnp.sum(e, axis=-1, keepdims=True)
        hatt = jnp.dot(att, v, preferred_element_type=_F32)
        hagg = h + jnp.maximum(_dgt(hatt, wna[...]) + vec_ref[vb + 3], 0.0)
        hagg = _ln(hagg, vec_ref[vb + 4], vec_ref[vb + 5])
        c = _dgt(h, wnc[...]) + vec_ref[vb + 6]
        wvn_a = wvn[:, :_H]
        wvn_b = wvn[:, _H:]
        hcom = hagg + jnp.maximum(
            _dgt(c, wvn_a) + _dgt(hagg, wvn_b) + vec_ref[vb + 7], 0.0)
        hn_next = _ln(hcom, vec_ref[vb + 8], vec_ref[vb + 9])
        e2refs[l][0] = (_dgt(h, we2[...])
                        + vec_ref[vb + 10]).reshape(_NT, _TI, _H)
        e3refs[l][0] = _dgt(h, we3[...]) + vec_ref[vb + 11]
        h = hn_next
    hn_ref[0] = h


def _edge_kernel(dist_ref, adj_ref, e2a_ref, e3a_ref, e2b_ref, e3b_ref,
                 wie_ref,
                 wea10_ref, wea0_ref, wec0_ref, wve0_ref,
                 wea11_ref, wea1_ref, wec1_ref, wve1_ref,
                 vec_ref, out_ref):
    d3 = dist_ref[0, 0][:, :, None]
    a3 = adj_ref[0, 0][:, :, None]
    u = vec_ref[0][None]
    z = vec_ref[1][None]
    bb = vec_ref[2][None]
    y = jnp.maximum(d3 * u + a3 * z + bb, 0.0).reshape(_R, _H)
    he = _dgt(y, wie_ref[...]) + vec_ref[3]
    mats = ((wea10_ref, wea0_ref, wec0_ref, wve0_ref),
            (wea11_ref, wea1_ref, wec1_ref, wve1_ref))
    e2refs = (e2a_ref, e2b_ref)
    e3refs = (e3a_ref, e3b_ref)
    for l in range(_L):
        wea1, wea, wec, wve = mats[l]
        vb = 4 + 8 * l
        e1 = _dgt(he, wea1[...]) + vec_ref[vb + 0]
        e2 = e2refs[l][0, 0]
        e3 = e3refs[l][0]
        s = (e1.reshape(_TI, _N, _H) + e2[:, None, :]
             + e3[None, :, :]).reshape(_R, _H)
        t = jnp.maximum(_dgt(s, wea[...]) + vec_ref[vb + 1], 0.0)
        hagg = _ln(he + t, vec_ref[vb + 2], vec_ref[vb + 3])
        c = _dgt(he, wec[...]) + vec_ref[vb + 4]
        wve_a = wve[:, :_H]
        wve_b = wve[:, _H:]
        hcom = hagg + jnp.maximum(
            _dgt(c, wve_a) + _dgt(hagg, wve_b) + vec_ref[vb + 5], 0.0)
        he = _ln(hcom, vec_ref[vb + 6], vec_ref[vb + 7])
    out_ref[0] = he.reshape(_TI, _N, _H)


def _full(shape):
    return pl.BlockSpec(shape, lambda b: (0,) * len(shape))


def _full2(shape):
    return pl.BlockSpec(shape, lambda b, t: (0,) * len(shape))


def _run(params, n_coor, n_info, dist, interpret=False):
    p = params
    lp0, lp1 = p["layers"]
    half = _H // 2
    zeros_h = jnp.zeros((half,), _F32)

    nvecs = [p["W1"]["b"],
             jnp.concatenate([p["W2"]["b"], p["W3"]["b"]]),
             p["init_n"]["b"]]
    for lp in (lp0, lp1):
        nvecs += [lp["attn_q"]["b"], lp["attn_k"]["b"], lp["attn_v"]["b"],
                  lp["W_node_agg"]["b"], lp["ln_na"]["g"], lp["ln_na"]["b"],
                  lp["V_node_com"]["b"], lp["V_node"]["b"],
                  lp["ln_nc"]["g"], lp["ln_nc"]["b"],
                  lp["W_edge_agg_2"]["b"], lp["W_edge_agg_3"]["b"]]
    nvecs = jnp.stack(nvecs)[:, None, :]

    evecs = [jnp.concatenate([p["W4"]["w"][:, 0], zeros_h]),
             jnp.concatenate([zeros_h, p["W5"]["w"][:, 0]]),
             jnp.concatenate([p["W4"]["b"], p["W5"]["b"]]),
             p["init_e"]["b"]]
    for lp in (lp0, lp1):
        evecs += [lp["W_edge_agg_1"]["b"], lp["W_edge_agg"]["b"],
                  lp["ln_ea"]["g"], lp["ln_ea"]["b"],
                  lp["V_edge_com"]["b"], lp["V_edge"]["b"],
                  lp["ln_ec"]["g"], lp["ln_ec"]["b"]]
    evecs = jnp.stack(evecs)[:, None, :]

    hh = pl.BlockSpec((1, _H, _H), lambda b: (0, 0, 0))
    mat_a = [pl.BlockSpec(p["W1"]["w"].shape, lambda b: (0, 0)),
             pl.BlockSpec(p["W2"]["w"].shape, lambda b: (0, 0)),
             pl.BlockSpec(p["W3"]["w"].shape, lambda b: (0, 0)),
             pl.BlockSpec((_H, _H), lambda b: (0, 0))]
    mats_a = [p["W1"]["w"], p["W2"]["w"], p["W3"]["w"], p["init_n"]["w"]]
    for lp in (lp0, lp1):
        mats_a += [lp["attn_q"]["w"], lp["attn_k"]["w"], lp["attn_v"]["w"],
                   lp["W_node_agg"]["w"], lp["V_node_com"]["w"],
                   lp["V_node"]["w"],
                   lp["W_edge_agg_2"]["w"], lp["W_edge_agg_3"]["w"]]
        mat_a += [pl.BlockSpec((_H, _H), lambda b: (0, 0))] * 5
        mat_a += [pl.BlockSpec((_H, 2 * _H), lambda b: (0, 0))]
        mat_a += [pl.BlockSpec((_H, _H), lambda b: (0, 0))] * 2

    b4 = pl.BlockSpec((1, _NT, _TI, _N), lambda b: (b, 0, 0, 0))
    e2spec = pl.BlockSpec((1, _NT, _TI, _H), lambda b: (b, 0, 0, 0))
    e3spec = pl.BlockSpec((1, _N, _H), lambda b: (b, 0, 0))
    adj, distr, hn, e2a, e3a, e2b, e3b = pl.pallas_call(
        _prep_kernel,
        grid=(_B,),
        in_specs=[
            pl.BlockSpec((1, _N, _N), lambda b: (b, 0, 0)),
            pl.BlockSpec((1, _N, 2), lambda b: (b, 0, 0)),
            pl.BlockSpec((1, _N, 3), lambda b: (b, 0, 0)),
            *mat_a,
            pl.BlockSpec(nvecs.shape, lambda b: (0, 0, 0)),
        ],
        out_specs=[b4, b4,
                   pl.BlockSpec((1, _N, _H), lambda b: (b, 0, 0)),
                   e2spec, e3spec, e2spec, e3spec],
        out_shape=[jax.ShapeDtypeStruct((_B, _NT, _TI, _N), _F32)] * 2
        + [jax.ShapeDtypeStruct((_B, _N, _H), _F32)]
        + [jax.ShapeDtypeStruct((_B, _NT, _TI, _H), _F32),
           jax.ShapeDtypeStruct((_B, _N, _H), _F32)] * 2,
        interpret=interpret,
    )(dist, n_coor, n_info, *mats_a, nvecs)

    mats_e = [p["init_e"]["w"]]
    mat_e = [pl.BlockSpec((_H, _H), lambda b, t: (0, 0))]
    for lp in (lp0, lp1):
        mats_e += [lp["W_edge_agg_1"]["w"], lp["W_edge_agg"]["w"],
                   lp["V_edge_com"]["w"], lp["V_edge"]["w"]]
        mat_e += [pl.BlockSpec((_H, _H), lambda b, t: (0, 0))] * 3
        mat_e += [pl.BlockSpec((_H, 2 * _H), lambda b, t: (0, 0))]

    he = pl.pallas_call(
        _edge_kernel,
        grid=(_B, _NT),
        in_specs=[
            pl.BlockSpec((1, 1, _TI, _N), lambda b, t: (b, t, 0, 0)),
            pl.BlockSpec((1, 1, _TI, _N), lambda b, t: (b, t, 0, 0)),
            pl.BlockSpec((1, 1, _TI, _H), lambda b, t: (b, t, 0, 0)),
            pl.BlockSpec((1, _N, _H), lambda b, t: (b, 0, 0)),
            pl.BlockSpec((1, 1, _TI, _H), lambda b, t: (b, t, 0, 0)),
            pl.BlockSpec((1, _N, _H), lambda b, t: (b, 0, 0)),
            *mat_e,
            pl.BlockSpec(evecs.shape, lambda b, t: (0, 0, 0)),
        ],
        out_specs=pl.BlockSpec((1, _TI, _N, _H), lambda b, t: (b, t, 0, 0)),
        out_shape=jax.ShapeDtypeStruct((_B, _N, _N, _H), _F32),
        interpret=interpret,
    )(distr, adj, e2a, e3a, e2b, e3b, *mats_e, evecs)

    return hn, he


def kernel(params, n_coor, n_info, dist):
    return _run(params, n_coor, n_info, dist)


# speedup vs baseline: 3.5705x; 1.0668x over previous
import math

import jax
import jax.numpy as jnp
from jax.experimental import pallas as pl

_B, _N, _H, _L, _K = 8, 100, 128, 2, 10
_TI = 20
_NT = _N // _TI
_R = _TI * _N
_F32 = jnp.float32


def _dgt(x, w):
    return jax.lax.dot_general(x, w, (((1,), (1,)), ((), ())),
                               preferred_element_type=_F32)


def _ln(x, g, b):
    m = jnp.mean(x, axis=-1, keepdims=True)
    d = x - m
    v = jnp.mean(d * d, axis=-1, keepdims=True)
    return d / jnp.sqrt(v + 1e-5) * g + b


def _ln_mxu(x, g, b):
    j = jnp.full((_H, _H), 1.0 / _H, _F32)
    m = jnp.dot(x, j, preferred_element_type=_F32)
    s2 = jnp.dot(x * x, j, preferred_element_type=_F32)
    return (x - m) * jax.lax.rsqrt(s2 - m * m + 1e-5) * g + b


def _prep_kernel(dist_ref, coor_ref, info_ref,
                 w1_ref, w2_ref, w3_ref, wi_ref,
                 wq0_ref, wk0_ref, wnav0_ref, wcn0_ref, wvn0_ref,
                 we20_ref, we30_ref,
                 wq1_ref, wk1_ref, wnav1_ref, wcn1_ref, wvn1_ref,
                 we21_ref, we31_ref,
                 vec_ref,
                 adj_ref, distr_ref, hn_ref,
                 e2a_ref, e3a_ref, e2b_ref, e3b_ref):
    w = dist_ref[0]
    col = jax.lax.broadcasted_iota(jnp.int32, (_N, _N), 1)
    row = jax.lax.broadcasted_iota(jnp.int32, (_N, _N), 0)
    adj = jnp.zeros((_N, _N), _F32)
    msk = jnp.zeros((_N, _N), _F32)
    for t in range(_K + 1):
        m = jnp.min(w, axis=-1, keepdims=True)
        cand = w == m
        idx = jnp.min(jnp.where(cand, col, jnp.int32(2**30)), axis=-1,
                      keepdims=True)
        hit = col == idx
        adj = adj + hit.astype(_F32)
        if t >= 1:
            msk = msk + hit.astype(_F32)
        w = jnp.where(hit, _F32(jnp.inf), w)
    adj = jnp.where(row == col, _F32(-1.0), adj)
    adj_ref[0] = adj.reshape(_NT, _TI, _N)
    distr_ref[0] = dist_ref[0].reshape(_NT, _TI, _N)

    coor = coor_ref[0]
    info = info_ref[0]
    x0 = jnp.maximum(_dgt(coor, w1_ref[...]) + vec_ref[0], 0.0)
    xi = jnp.maximum(
        jnp.concatenate([_dgt(coor, w2_ref[...]), _dgt(info, w3_ref[...])],
                        axis=-1) + vec_ref[1], 0.0)
    rid = jax.lax.broadcasted_iota(jnp.int32, (_N, _H), 0)
    x = jnp.where(rid == 0, x0, xi)
    h = _dgt(x, wi_ref[...]) + vec_ref[2]
    mats = ((wq0_ref, wk0_ref, wnav0_ref, wcn0_ref, wvn0_ref,
             we20_ref, we30_ref),
            (wq1_ref, wk1_ref, wnav1_ref, wcn1_ref, wvn1_ref,
             we21_ref, we31_ref))
    e2refs = (e2a_ref, e2b_ref)
    e3refs = (e3a_ref, e3b_ref)
    for l in range(_L):
        wq, wk, wnav, wcn, wvn, we2, we3 = mats[l]
        vb = 3 + 10 * l
        q = _dgt(h, wq[...]) + vec_ref[vb + 0]
        k = _dgt(h, wk[...]) + vec_ref[vb + 1]
        s = _dgt(q, k) * (1.0 / math.sqrt(_H))
        s = jnp.where(msk > 0.5, s, _F32(-1e30))
        smax = jnp.max(s, axis=-1, keepdims=True)
        e = jnp.exp(s - smax)
        att = e / jnp.sum(e, axis=-1, keepdims=True)
        vprime = _dgt(h, wnav[...])
        hagg = h + jnp.maximum(
            jnp.dot(att, vprime, preferred_element_type=_F32)
            + vec_ref[vb + 2], 0.0)
        hagg = _ln(hagg, vec_ref[vb + 3], vec_ref[vb + 4])
        hcom = hagg + jnp.maximum(
            _dgt(h, wcn[...]) + _dgt(hagg, wvn[:, _H:]) + vec_ref[vb + 5],
            0.0)
        hn_next = _ln(hcom, vec_ref[vb + 6], vec_ref[vb + 7])
        e2refs[l][0] = (_dgt(h, we2[...])
                        + vec_ref[vb + 8]).reshape(_NT, _TI, _H)
        e3refs[l][0] = _dgt(h, we3[...]) + vec_ref[vb + 9]
        h = hn_next
    hn_ref[0] = h


def _edge_kernel(dist_ref, adj_ref, e2a_ref, e3a_ref, e2b_ref, e3b_ref,
                 wie_ref,
                 wea10_ref, wea0_ref, wce0_ref, wve0_ref,
                 wea11_ref, wea1_ref, wce1_ref, wve1_ref,
                 vec_ref, out_ref):
    d3 = dist_ref[0, 0][:, :, None]
    a3 = adj_ref[0, 0][:, :, None]
    u = vec_ref[0][None]
    z = vec_ref[1][None]
    bb = vec_ref[2][None]
    y = jnp.maximum(d3 * u + a3 * z + bb, 0.0).reshape(_R, _H)
    he = _dgt(y, wie_ref[...]) + vec_ref[3]
    mats = ((wea10_ref, wea0_ref, wce0_ref, wve0_ref),
            (wea11_ref, wea1_ref, wce1_ref, wve1_ref))
    e2refs = (e2a_ref, e2b_ref)
    e3refs = (e3a_ref, e3b_ref)
    for l in range(_L):
        wea1, wea, wce, wve = mats[l]
        vb = 4 + 6 * l
        e1 = _dgt(he, wea1[...])
        e2 = e2refs[l][0, 0]
        e3 = e3refs[l][0]
        s = (e1.reshape(_TI, _N, _H) + e2[:, None, :]
             + e3[None, :, :]).reshape(_R, _H)
        t = jnp.maximum(_dgt(s, wea[...]) + vec_ref[vb + 0], 0.0)
        hagg = _ln_mxu(he + t, vec_ref[vb + 1], vec_ref[vb + 2])
        hcom = hagg + jnp.maximum(
            _dgt(he, wce[...]) + _dgt(hagg, wve[:, _H:]) + vec_ref[vb + 3],
            0.0)
        he = _ln_mxu(hcom, vec_ref[vb + 4], vec_ref[vb + 5])
    out_ref[0] = he.reshape(_TI, _N, _H)


def _run(params, n_coor, n_info, dist, interpret=False):
    p = params
    lp0, lp1 = p["layers"]
    half = _H // 2
    zeros_h = jnp.zeros((half,), _F32)

    nvecs = [p["W1"]["b"],
             jnp.concatenate([p["W2"]["b"], p["W3"]["b"]]),
             p["init_n"]["b"]]
    nmats = [p["W1"]["w"], p["W2"]["w"], p["W3"]["w"], p["init_n"]["w"]]
    for lp in (lp0, lp1):
        wvn_a = lp["V_node"]["w"][:, :_H]
        wna = lp["W_node_agg"]["w"]
        nmats += [lp["attn_q"]["w"], lp["attn_k"]["w"],
                  wna @ lp["attn_v"]["w"],
                  wvn_a @ lp["V_node_com"]["w"],
                  lp["V_node"]["w"],
                  lp["W_edge_agg_2"]["w"], lp["W_edge_agg_3"]["w"]]
        nvecs += [lp["attn_q"]["b"], lp["attn_k"]["b"],
                  wna @ lp["attn_v"]["b"] + lp["W_node_agg"]["b"],
                  lp["ln_na"]["g"], lp["ln_na"]["b"],
                  wvn_a @ lp["V_node_com"]["b"] + lp["V_node"]["b"],
                  lp["ln_nc"]["g"], lp["ln_nc"]["b"],
                  lp["W_edge_agg_2"]["b"] + lp["W_edge_agg_1"]["b"],
                  lp["W_edge_agg_3"]["b"]]
    nvecs = jnp.stack(nvecs)[:, None, :]

    evecs = [jnp.concatenate([p["W4"]["w"][:, 0], zeros_h]),
             jnp.concatenate([zeros_h, p["W5"]["w"][:, 0]]),
             jnp.concatenate([p["W4"]["b"], p["W5"]["b"]]),
             p["init_e"]["b"]]
    emats = [p["init_e"]["w"]]
    for lp in (lp0, lp1):
        wve_a = lp["V_edge"]["w"][:, :_H]
        emats += [lp["W_edge_agg_1"]["w"], lp["W_edge_agg"]["w"],
                  wve_a @ lp["V_edge_com"]["w"], lp["V_edge"]["w"]]
        evecs += [lp["W_edge_agg"]["b"],
                  lp["ln_ea"]["g"], lp["ln_ea"]["b"],
                  wve_a @ lp["V_edge_com"]["b"] + lp["V_edge"]["b"],
                  lp["ln_ec"]["g"], lp["ln_ec"]["b"]]
    evecs = jnp.stack(evecs)[:, None, :]

    mat_a = [pl.BlockSpec(m.shape, lambda b: (0, 0)) for m in nmats]
    b4 = pl.BlockSpec((1, _NT, _TI, _N), lambda b: (b, 0, 0, 0))
    e2spec = pl.BlockSpec((1, _NT, _TI, _H), lambda b: (b, 0, 0, 0))
    e3spec = pl.BlockSpec((1, _N, _H), lambda b: (b, 0, 0))
    adj, distr, hn, e2a, e3a, e2b, e3b = pl.pallas_call(
        _prep_kernel,
        grid=(_B,),
        in_specs=[
            pl.BlockSpec((1, _N, _N), lambda b: (b, 0, 0)),
            pl.BlockSpec((1, _N, 2), lambda b: (b, 0, 0)),
            pl.BlockSpec((1, _N, 3), lambda b: (b, 0, 0)),
            *mat_a,
            pl.BlockSpec(nvecs.shape, lambda b: (0, 0, 0)),
        ],
        out_specs=[b4, b4,
                   pl.BlockSpec((1, _N, _H), lambda b: (b, 0, 0)),
                   e2spec, e3spec, e2spec, e3spec],
        out_shape=[jax.ShapeDtypeStruct((_B, _NT, _TI, _N), _F32)] * 2
        + [jax.ShapeDtypeStruct((_B, _N, _H), _F32)]
        + [jax.ShapeDtypeStruct((_B, _NT, _TI, _H), _F32),
           jax.ShapeDtypeStruct((_B, _N, _H), _F32)] * 2,
        interpret=interpret,
    )(dist, n_coor, n_info, *nmats, nvecs)

    mat_e = [pl.BlockSpec(m.shape, lambda b, t: (0, 0)) for m in emats]
    he = pl.pallas_call(
        _edge_kernel,
        grid=(_B, _NT),
        in_specs=[
            pl.BlockSpec((1, 1, _TI, _N), lambda b, t: (b, t, 0, 0)),
            pl.BlockSpec((1, 1, _TI, _N), lambda b, t: (b, t, 0, 0)),
            pl.BlockSpec((1, 1, _TI, _H), lambda b, t: (b, t, 0, 0)),
            pl.BlockSpec((1, _N, _H), lambda b, t: (b, 0, 0)),
            pl.BlockSpec((1, 1, _TI, _H), lambda b, t: (b, t, 0, 0)),
            pl.BlockSpec((1, _N, _H), lambda b, t: (b, 0, 0)),
            *mat_e,
            pl.BlockSpec(evecs.shape, lambda b, t: (0, 0, 0)),
        ],
        out_specs=pl.BlockSpec((1, _TI, _N, _H), lambda b, t: (b, t, 0, 0)),
        out_shape=jax.ShapeDtypeStruct((_B, _N, _N, _H), _F32),
        interpret=interpret,
    )(distr, adj, e2a, e3a, e2b, e3b, *emats, evecs)

    return hn, he


def kernel(params, n_coor, n_info, dist):
    return _run(params, n_coor, n_info, dist)


# speedup vs baseline: 3.7564x; 1.0521x over previous
import math

import jax
import jax.numpy as jnp
from jax.experimental import pallas as pl

_B, _N, _H, _L, _K = 8, 100, 128, 2, 10
_TI = 20
_NT = _N // _TI
_R = _TI * _N
_F32 = jnp.float32


def _dgt(x, w):
    return jax.lax.dot_general(x, w, (((1,), (1,)), ((), ())),
                               preferred_element_type=_F32)


def _ln(x, g, b):
    m = jnp.mean(x, axis=-1, keepdims=True)
    d = x - m
    v = jnp.mean(d * d, axis=-1, keepdims=True)
    return d / jnp.sqrt(v + 1e-5) * g + b


def _ln_mxu(x, g, b):
    j = jnp.full((_H, _H), 1.0 / _H, _F32)
    m = jnp.dot(x, j, preferred_element_type=_F32)
    s2 = jnp.dot(x * x, j, preferred_element_type=_F32)
    return (x - m) * jax.lax.rsqrt(s2 - m * m + 1e-5) * g + b


def _prep_kernel(dist_ref, coor_ref, info_ref,
                 w1_ref, w2_ref, w3_ref, wi_ref,
                 wq0_ref, wk0_ref, wna0_ref, wv0_ref, wnc0_ref, wvn0_ref,
                 we20_ref, we30_ref,
                 wq1_ref, wk1_ref, wna1_ref, wv1_ref, wnc1_ref, wvn1_ref,
                 we21_ref, we31_ref,
                 vec_ref,
                 adj_ref, distr_ref, hn_ref,
                 e2a_ref, e3a_ref, e2b_ref, e3b_ref):
    w = dist_ref[0]
    col = jax.lax.broadcasted_iota(jnp.int32, (_N, _N), 1)
    row = jax.lax.broadcasted_iota(jnp.int32, (_N, _N), 0)
    adj = jnp.zeros((_N, _N), _F32)
    msk = jnp.zeros((_N, _N), _F32)
    for t in range(_K + 1):
        m = jnp.min(w, axis=-1, keepdims=True)
        cand = w == m
        idx = jnp.min(jnp.where(cand, col, jnp.int32(2**30)), axis=-1,
                      keepdims=True)
        hit = col == idx
        adj = adj + hit.astype(_F32)
        if t >= 1:
            msk = msk + hit.astype(_F32)
        w = jnp.where(hit, _F32(jnp.inf), w)
    adj = jnp.where(row == col, _F32(-1.0), adj)
    adj_ref[0] = adj.reshape(_NT, _TI, _N)
    distr_ref[0] = dist_ref[0].reshape(_NT, _TI, _N)

    coor = coor_ref[0]
    info = info_ref[0]
    x0 = jnp.maximum(_dgt(coor, w1_ref[...]) + vec_ref[0], 0.0)
    xi = jnp.maximum(
        jnp.concatenate([_dgt(coor, w2_ref[...]), _dgt(info, w3_ref[...])],
                        axis=-1) + vec_ref[1], 0.0)
    rid = jax.lax.broadcasted_iota(jnp.int32, (_N, _H), 0)
    x = jnp.where(rid == 0, x0, xi)
    h = _dgt(x, wi_ref[...]) + vec_ref[2]
    mats = ((wq0_ref, wk0_ref, wna0_ref, wv0_ref, wnc0_ref, wvn0_ref,
             we20_ref, we30_ref),
            (wq1_ref, wk1_ref, wna1_ref, wv1_ref, wnc1_ref, wvn1_ref,
             we21_ref, we31_ref))
    e2refs = (e2a_ref, e2b_ref)
    e3refs = (e3a_ref, e3b_ref)
    for l in range(_L):
        wq, wk, wna, wv, wnc, wvn, we2, we3 = mats[l]
        wnav = jnp.dot(wna[...], wv[...], preferred_element_type=_F32)
        wcn = jnp.dot(wvn[:, :_H], wnc[...], preferred_element_type=_F32)
        vb = 3 + 12 * l
        q = _dgt(h, wq[...]) + vec_ref[vb + 0]
        k = _dgt(h, wk[...]) + vec_ref[vb + 1]
        s = _dgt(q, k) * (1.0 / math.sqrt(_H))
        s = jnp.where(msk > 0.5, s, _F32(-1e30))
        smax = jnp.max(s, axis=-1, keepdims=True)
        e = jnp.exp(s - smax)
        att = e / jnp.sum(e, axis=-1, keepdims=True)
        vprime = _dgt(h, wnav)
        battn = _dgt(vec_ref[vb + 2], wna[...]) + vec_ref[vb + 3]
        hagg = h + jnp.maximum(
            jnp.dot(att, vprime, preferred_element_type=_F32)
            + battn, 0.0)
        hagg = _ln(hagg, vec_ref[vb + 4], vec_ref[vb + 5])
        bcn = _dgt(vec_ref[vb + 6], wvn[:, :_H]) + vec_ref[vb + 7]
        hcom = hagg + jnp.maximum(
            _dgt(h, wcn) + _dgt(hagg, wvn[:, _H:]) + bcn,
            0.0)
        hn_next = _ln(hcom, vec_ref[vb + 8], vec_ref[vb + 9])
        e2refs[l][0] = (_dgt(h, we2[...])
                        + vec_ref[vb + 10]).reshape(_NT, _TI, _H)
        e3refs[l][0] = _dgt(h, we3[...]) + vec_ref[vb + 11]
        h = hn_next
    hn_ref[0] = h


def _edge_kernel(dist_ref, adj_ref, e2a_ref, e3a_ref, e2b_ref, e3b_ref,
                 wie_ref,
                 wea10_ref, wea0_ref, wce0_ref, wve0_ref,
                 wea11_ref, wea1_ref, wce1_ref, wve1_ref,
                 vec_ref, out_ref):
    d3 = dist_ref[0, 0][:, :, None]
    a3 = adj_ref[0, 0][:, :, None]
    u = vec_ref[0][None]
    z = vec_ref[1][None]
    bb = vec_ref[2][None]
    y = jnp.maximum(d3 * u + a3 * z + bb, 0.0).reshape(_R, _H)
    he = _dgt(y, wie_ref[...]) + vec_ref[3]
    mats = ((wea10_ref, wea0_ref, wce0_ref, wve0_ref),
            (wea11_ref, wea1_ref, wce1_ref, wve1_ref))
    e2refs = (e2a_ref, e2b_ref)
    e3refs = (e3a_ref, e3b_ref)
    for l in range(_L):
        wea1, wea, wec, wve = mats[l]
        wce = jnp.dot(wve[:, :_H], wec[...], preferred_element_type=_F32)
        vb = 4 + 7 * l
        e1 = _dgt(he, wea1[...])
        e2 = e2refs[l][0, 0]
        e3 = e3refs[l][0]
        s = (e1.reshape(_TI, _N, _H) + e2[:, None, :]
             + e3[None, :, :]).reshape(_R, _H)
        t = jnp.maximum(_dgt(s, wea[...]) + vec_ref[vb + 0], 0.0)
        hagg = _ln_mxu(he + t, vec_ref[vb + 1], vec_ref[vb + 2])
        bce = _dgt(vec_ref[vb + 3], wve[:, :_H]) + vec_ref[vb + 4]
        hcom = hagg + jnp.maximum(
            _dgt(he, wce) + _dgt(hagg, wve[:, _H:]) + bce,
            0.0)
        he = _ln_mxu(hcom, vec_ref[vb + 5], vec_ref[vb + 6])
    out_ref[0] = he.reshape(_TI, _N, _H)


def _run(params, n_coor, n_info, dist, interpret=False):
    p = params
    lp0, lp1 = p["layers"]
    half = _H // 2
    zeros_h = jnp.zeros((half,), _F32)

    nvecs = [p["W1"]["b"],
             jnp.concatenate([p["W2"]["b"], p["W3"]["b"]]),
             p["init_n"]["b"]]
    nmats = [p["W1"]["w"], p["W2"]["w"], p["W3"]["w"], p["init_n"]["w"]]
    for lp in (lp0, lp1):
        nmats += [lp["attn_q"]["w"], lp["attn_k"]["w"],
                  lp["W_node_agg"]["w"], lp["attn_v"]["w"],
                  lp["V_node_com"]["w"], lp["V_node"]["w"],
                  lp["W_edge_agg_2"]["w"], lp["W_edge_agg_3"]["w"]]
        nvecs += [lp["attn_q"]["b"], lp["attn_k"]["b"],
                  lp["attn_v"]["b"], lp["W_node_agg"]["b"],
                  lp["ln_na"]["g"], lp["ln_na"]["b"],
                  lp["V_node_com"]["b"], lp["V_node"]["b"],
                  lp["ln_nc"]["g"], lp["ln_nc"]["b"],
                  lp["W_edge_agg_2"]["b"] + lp["W_edge_agg_1"]["b"],
                  lp["W_edge_agg_3"]["b"]]
    nvecs = jnp.stack(nvecs)[:, None, :]

    evecs = [jnp.concatenate([p["W4"]["w"][:, 0], zeros_h]),
             jnp.concatenate([zeros_h, p["W5"]["w"][:, 0]]),
             jnp.concatenate([p["W4"]["b"], p["W5"]["b"]]),
             p["init_e"]["b"]]
    emats = [p["init_e"]["w"]]
    for lp in (lp0, lp1):
        emats += [lp["W_edge_agg_1"]["w"], lp["W_edge_agg"]["w"],
                  lp["V_edge_com"]["w"], lp["V_edge"]["w"]]
        evecs += [lp["W_edge_agg"]["b"],
                  lp["ln_ea"]["g"], lp["ln_ea"]["b"],
                  lp["V_edge_com"]["b"], lp["V_edge"]["b"],
                  lp["ln_ec"]["g"], lp["ln_ec"]["b"]]
    evecs = jnp.stack(evecs)[:, None, :]

    mat_a = [pl.BlockSpec(m.shape, lambda b: (0, 0)) for m in nmats]
    b4 = pl.BlockSpec((1, _NT, _TI, _N), lambda b: (b, 0, 0, 0))
    e2spec = pl.BlockSpec((1, _NT, _TI, _H), lambda b: (b, 0, 0, 0))
    e3spec = pl.BlockSpec((1, _N, _H), lambda b: (b, 0, 0))
    adj, distr, hn, e2a, e3a, e2b, e3b = pl.pallas_call(
        _prep_kernel,
        grid=(_B,),
        in_specs=[
            pl.BlockSpec((1, _N, _N), lambda b: (b, 0, 0)),
            pl.BlockSpec((1, _N, 2), lambda b: (b, 0, 0)),
            pl.BlockSpec((1, _N, 3), lambda b: (b, 0, 0)),
            *mat_a,
            pl.BlockSpec(nvecs.shape, lambda b: (0, 0, 0)),
        ],
        out_specs=[b4, b4,
                   pl.BlockSpec((1, _N, _H), lambda b: (b, 0, 0)),
                   e2spec, e3spec, e2spec, e3spec],
        out_shape=[jax.ShapeDtypeStruct((_B, _NT, _TI, _N), _F32)] * 2
        + [jax.ShapeDtypeStruct((_B, _N, _H), _F32)]
        + [jax.ShapeDtypeStruct((_B, _NT, _TI, _H), _F32),
           jax.ShapeDtypeStruct((_B, _N, _H), _F32)] * 2,
        interpret=interpret,
    )(dist, n_coor, n_info, *nmats, nvecs)

    mat_e = [pl.BlockSpec(m.shape, lambda b, t: (0, 0)) for m in emats]
    he = pl.pallas_call(
        _edge_kernel,
        grid=(_B, _NT),
        in_specs=[
            pl.BlockSpec((1, 1, _TI, _N), lambda b, t: (b, t, 0, 0)),
            pl.BlockSpec((1, 1, _TI, _N), lambda b, t: (b, t, 0, 0)),
            pl.BlockSpec((1, 1, _TI, _H), lambda b, t: (b, t, 0, 0)),
            pl.BlockSpec((1, _N, _H), lambda b, t: (b, 0, 0)),
            pl.BlockSpec((1, 1, _TI, _H), lambda b, t: (b, t, 0, 0)),
            pl.BlockSpec((1, _N, _H), lambda b, t: (b, 0, 0)),
            *mat_e,
            pl.BlockSpec(evecs.shape, lambda b, t: (0, 0, 0)),
        ],
        out_specs=pl.BlockSpec((1, _TI, _N, _H), lambda b, t: (b, t, 0, 0)),
        out_shape=jax.ShapeDtypeStruct((_B, _N, _N, _H), _F32),
        interpret=interpret,
    )(distr, adj, e2a, e3a, e2b, e3b, *emats, evecs)

    return hn, he


def kernel(params, n_coor, n_info, dist):
    return _run(params, n_coor, n_info, dist)
